# baseline (device time: 84681 ns/iter reference)
import jax
import jax.numpy as jnp
from jax import lax
from jax.experimental import pallas as pl
from jax.experimental.pallas import tpu as pltpu

C = 16
T = 3
H = C // 2
D = C + 2 * T


def kernel(x, pi):
    _, m, n = x.shape
    quarter = m // 4
    rows = quarter // C

    def body(x_ref, pi_ref, out_ref, stage, sendbuf, copy_sems,
             dsend, drecv, xqsend, xqrecv, zqsend, zqrecv,
             xdsend, xdrecv, zdsend, zdrecv):
        my_x = lax.axis_index("x")
        my_y = lax.axis_index("y")
        my_z = lax.axis_index("z")
        tgt_y = jnp.where(my_y == 0, pi_ref[0], pi_ref[1])

        qoff_me = (2 * my_x + my_z) * quarter
        qoff_x = (2 * (1 - my_x) + my_z) * quarter
        qoff_z = (2 * my_x + (1 - my_z)) * quarter

        x_nbr = (1 - my_x, my_y, my_z)
        z_nbr = (my_x, my_y, 1 - my_z)

        barrier_sem = pltpu.get_barrier_semaphore()
        for nbr in [(my_x, tgt_y, my_z), x_nbr, z_nbr]:
            pl.semaphore_signal(
                barrier_sem, inc=1,
                device_id=nbr, device_id_type=pl.DeviceIdType.MESH,
            )
        pl.semaphore_wait(barrier_sem, 3)

        def remote(src_rows_, dst_rows_, ssem, rsem, dev):
            rd = pltpu.make_async_remote_copy(
                src_ref=out_ref.at[:, pl.ds(src_rows_, rows), :],
                dst_ref=out_ref.at[:, pl.ds(dst_rows_, rows), :],
                send_sem=ssem,
                recv_sem=rsem,
                device_id=dev,
                device_id_type=pl.DeviceIdType.MESH,
            )
            rd.start()
            return rd

        src_rows = [qoff_me + c * rows for c in range(C - T)]
        src_rows += [qoff_x + (C - T + j) * rows for j in range(T)]
        src_rows += [qoff_z + (C - T + j) * rows for j in range(T)]
        src_rows += [qoff_me + (C - T + j) * rows for j in range(T)]
        i_xtail = C - T
        i_ztail = C
        i_own = C + T

        def start_copy(c):
            cp = pltpu.make_async_copy(
                x_ref.at[:, pl.ds(src_rows[c], rows), :],
                stage.at[c % 2],
                copy_sems.at[c % 2],
            )
            cp.start()
            return cp

        direct = []
        cp = start_copy(0)
        for c in range(D):
            nxt = start_copy(c + 1) if c + 1 < D else None
            cp.wait()
            sl = pl.ds(c * rows, rows)
            sendbuf[:, sl, :] = stage[c % 2].astype(jnp.bfloat16)
            rd = pltpu.make_async_remote_copy(
                src_ref=sendbuf.at[:, sl, :],
                dst_ref=out_ref.at[:, pl.ds(src_rows[c], rows), :],
                send_sem=dsend.at[c],
                recv_sem=drecv.at[c],
                device_id=(my_x, tgt_y, my_z),
                device_id_type=pl.DeviceIdType.MESH,
            )
            rd.start()
            direct.append(rd)
            cp = nxt

        xq, zq = [], []
        for c in range(C - T):
            direct[c].wait()
            r = qoff_me + c * rows
            xq.append(remote(r, r, xqsend.at[c], xqrecv.at[c], x_nbr))
            zq.append(remote(r, r, zqsend.at[c], zqrecv.at[c], z_nbr))

        xd, zd = [], []
        for i in range(H):
            zq[i].wait_recv()
            r = qoff_z + i * rows
            xd.append(remote(r, r, xdsend.at[i], xdrecv.at[i], x_nbr))
            c = H + i
            if c < C - T:
                xq[c].wait_recv()
            else:
                direct[i_xtail + (c - (C - T))].wait()
            r = qoff_x + c * rows
            zd.append(remote(r, r, zdsend.at[i], zdrecv.at[i], z_nbr))

        for j in range(T):
            direct[i_ztail + j].wait()
            direct[i_own + j].wait()
        for i in range(H):
            xq[i].wait_recv()
        for i in range(H, C - T):
            zq[i].wait_recv()
        for c in range(C - T):
            xq[c].wait_send()
            zq[c].wait_send()
        for i in range(H):
            xd[i].wait()
            zd[i].wait()

    return pl.pallas_call(
        body,
        out_shape=jax.ShapeDtypeStruct(x.shape, jnp.bfloat16),
        in_specs=[
            pl.BlockSpec(memory_space=pl.ANY),
            pl.BlockSpec(memory_space=pltpu.SMEM),
        ],
        out_specs=pl.BlockSpec(memory_space=pl.ANY),
        scratch_shapes=[
            pltpu.VMEM((2, 1, rows, n), jnp.float32),
            pltpu.VMEM((1, D * rows, n), jnp.bfloat16),
            pltpu.SemaphoreType.DMA((2,)),
            pltpu.SemaphoreType.DMA((D,)),
            pltpu.SemaphoreType.DMA((D,)),
            pltpu.SemaphoreType.DMA((C - T,)),
            pltpu.SemaphoreType.DMA((C - T,)),
            pltpu.SemaphoreType.DMA((C - T,)),
            pltpu.SemaphoreType.DMA((C - T,)),
            pltpu.SemaphoreType.DMA((H,)),
            pltpu.SemaphoreType.DMA((H,)),
            pltpu.SemaphoreType.DMA((H,)),
            pltpu.SemaphoreType.DMA((H,)),
        ],
        compiler_params=pltpu.CompilerParams(collective_id=0),
    )(x, pi)


# device time: 83108 ns/iter; 1.0189x vs baseline; 1.0189x over previous
import jax
import jax.numpy as jnp
from jax import lax
from jax.experimental import pallas as pl
from jax.experimental.pallas import tpu as pltpu

C = 8
T = 1
H = C // 2
D = C + 2 * T


def kernel(x, pi):
    _, m, n = x.shape
    quarter = m // 4
    rows = quarter // C

    def body(x_ref, pi_ref, out_ref, stage, sendbuf, copy_sems,
             dsend, drecv, xqsend, xqrecv, zqsend, zqrecv,
             xdsend, xdrecv, zdsend, zdrecv):
        my_x = lax.axis_index("x")
        my_y = lax.axis_index("y")
        my_z = lax.axis_index("z")
        tgt_y = jnp.where(my_y == 0, pi_ref[0], pi_ref[1])

        qoff_me = (2 * my_x + my_z) * quarter
        qoff_x = (2 * (1 - my_x) + my_z) * quarter
        qoff_z = (2 * my_x + (1 - my_z)) * quarter

        x_nbr = (1 - my_x, my_y, my_z)
        z_nbr = (my_x, my_y, 1 - my_z)

        barrier_sem = pltpu.get_barrier_semaphore()
        for nbr in [(my_x, tgt_y, my_z), x_nbr, z_nbr]:
            pl.semaphore_signal(
                barrier_sem, inc=1,
                device_id=nbr, device_id_type=pl.DeviceIdType.MESH,
            )
        pl.semaphore_wait(barrier_sem, 3)

        def remote(src_rows_, dst_rows_, ssem, rsem, dev):
            rd = pltpu.make_async_remote_copy(
                src_ref=out_ref.at[:, pl.ds(src_rows_, rows), :],
                dst_ref=out_ref.at[:, pl.ds(dst_rows_, rows), :],
                send_sem=ssem,
                recv_sem=rsem,
                device_id=dev,
                device_id_type=pl.DeviceIdType.MESH,
            )
            rd.start()
            return rd

        src_rows = [qoff_me + c * rows for c in range(C - T)]
        src_rows += [qoff_x + (C - T + j) * rows for j in range(T)]
        src_rows += [qoff_z + (C - T + j) * rows for j in range(T)]
        src_rows += [qoff_me + (C - T + j) * rows for j in range(T)]
        i_xtail = C - T
        i_ztail = C
        i_own = C + T

        def start_copy(c):
            cp = pltpu.make_async_copy(
                x_ref.at[:, pl.ds(src_rows[c], rows), :],
                stage.at[c % 2],
                copy_sems.at[c % 2],
            )
            cp.start()
            return cp

        direct = []
        cp = start_copy(0)
        for c in range(D):
            nxt = start_copy(c + 1) if c + 1 < D else None
            cp.wait()
            sl = pl.ds(c * rows, rows)
            sendbuf[:, sl, :] = stage[c % 2].astype(jnp.bfloat16)
            rd = pltpu.make_async_remote_copy(
                src_ref=sendbuf.at[:, sl, :],
                dst_ref=out_ref.at[:, pl.ds(src_rows[c], rows), :],
                send_sem=dsend.at[c],
                recv_sem=drecv.at[c],
                device_id=(my_x, tgt_y, my_z),
                device_id_type=pl.DeviceIdType.MESH,
            )
            rd.start()
            direct.append(rd)
            cp = nxt

        xq, zq = [], []
        for c in range(C - T):
            direct[c].wait()
            r = qoff_me + c * rows
            xq.append(remote(r, r, xqsend.at[c], xqrecv.at[c], x_nbr))
            zq.append(remote(r, r, zqsend.at[c], zqrecv.at[c], z_nbr))

        xd, zd = [], []
        for i in range(H):
            zq[i].wait_recv()
            r = qoff_z + i * rows
            xd.append(remote(r, r, xdsend.at[i], xdrecv.at[i], x_nbr))
            c = H + i
            if c < C - T:
                xq[c].wait_recv()
            else:
                direct[i_xtail + (c - (C - T))].wait()
            r = qoff_x + c * rows
            zd.append(remote(r, r, zdsend.at[i], zdrecv.at[i], z_nbr))

        for j in range(T):
            direct[i_ztail + j].wait()
            direct[i_own + j].wait()
        for i in range(H):
            xq[i].wait_recv()
        for i in range(H, C - T):
            zq[i].wait_recv()
        for c in range(C - T):
            xq[c].wait_send()
            zq[c].wait_send()
        for i in range(H):
            xd[i].wait()
            zd[i].wait()

    return pl.pallas_call(
        body,
        out_shape=jax.ShapeDtypeStruct(x.shape, jnp.bfloat16),
        in_specs=[
            pl.BlockSpec(memory_space=pl.ANY),
            pl.BlockSpec(memory_space=pltpu.SMEM),
        ],
        out_specs=pl.BlockSpec(memory_space=pl.ANY),
        scratch_shapes=[
            pltpu.VMEM((2, 1, rows, n), jnp.float32),
            pltpu.VMEM((1, D * rows, n), jnp.bfloat16),
            pltpu.SemaphoreType.DMA((2,)),
            pltpu.SemaphoreType.DMA((D,)),
            pltpu.SemaphoreType.DMA((D,)),
            pltpu.SemaphoreType.DMA((C - T,)),
            pltpu.SemaphoreType.DMA((C - T,)),
            pltpu.SemaphoreType.DMA((C - T,)),
            pltpu.SemaphoreType.DMA((C - T,)),
            pltpu.SemaphoreType.DMA((H,)),
            pltpu.SemaphoreType.DMA((H,)),
            pltpu.SemaphoreType.DMA((H,)),
            pltpu.SemaphoreType.DMA((H,)),
        ],
        compiler_params=pltpu.CompilerParams(collective_id=0),
    )(x, pi)
